# baseline (speedup 1.0000x reference)
import functools

import jax
import jax.numpy as jnp
from jax import lax
from jax.experimental import pallas as pl
from jax.experimental.pallas import tpu as pltpu
from jax.experimental.pallas import tpu_sc as plsc

N = 10000
E = 320000
D = 128
H1 = 256
H2 = 256
H3 = 128

NC, NS, LANES = 2, 16, 16
NW = NC * NS
CHUNK = 128
PHASES = 2
CPP = 40
CHUNKS_PW = PHASES * CPP
E_PAD = NW * CHUNKS_PW * CHUNK
PAD_ROWS = 112
N_ACC = N + PAD_ROWS
ROWS_PER_SUB = 624

_mesh = plsc.VectorSubcoreMesh(core_axis_name="c", subcore_axis_name="s")


def _make_agg(with_count):
    def body(table, srcr, dstr, z2, z1, out_acc, out_cnt,
             idx_s, idx_d, rows, ones_v, acc, cnt, gsem0, gsem1):
        c = lax.axis_index("c")
        s = lax.axis_index("s")
        w = s * NC + c
        gsems = (gsem0, gsem1)

        if with_count:
            for k in range(CHUNK // LANES):
                ones_v[pl.ds(k * LANES, LANES)] = jnp.ones((LANES,),
                                                           jnp.float32)

        rps = N_ACC // NS
        pltpu.sync_copy(z2.at[pl.ds(s * rps, rps)],
                        acc.at[pl.ds(s * rps, rps)])

        @pl.when(s == 0)
        def _():
            pltpu.sync_copy(z1, cnt)

        plsc.subcore_barrier()

        def _gather_start(cj, b):
            pltpu.async_copy(table.at[idx_s.at[cj]], rows.at[b], gsems[b])

        def _gather_wait(b):
            pltpu.make_async_copy(table.at[idx_s.at[0]], rows.at[b],
                                  gsems[b]).wait()

        for p in range(PHASES):
            pltpu.sync_copy(srcr.at[w, pl.ds(p * CPP, CPP)],
                            idx_s.at[pl.ds(0, CPP)])
            nxt = (p + 1) * CPP if p + 1 < PHASES else 0
            pltpu.sync_copy(srcr.at[w, nxt], idx_s.at[CPP])
            pltpu.sync_copy(dstr.at[w, pl.ds(p * CPP, CPP)], idx_d)

            _gather_start(0, 0)

            def loop(j, carry):
                for b in range(4):
                    cj = 4 * j + b
                    _gather_start(cj + 1, 1 - b % 2)
                    if with_count:
                        pltpu.sync_copy(ones_v, cnt.at[idx_d.at[cj]],
                                        add=True)
                    _gather_wait(b % 2)
                    pltpu.sync_copy(rows.at[b % 2], acc.at[idx_d.at[cj]],
                                    add=True)
                return carry

            lax.fori_loop(0, CPP // 4, loop, 0)
            _gather_wait(0)

        plsc.subcore_barrier()

        pltpu.sync_copy(acc.at[pl.ds(s * ROWS_PER_SUB, ROWS_PER_SUB)],
                        out_acc.at[c, pl.ds(s * ROWS_PER_SUB, ROWS_PER_SUB)])

        @pl.when(s == 0)
        def _():
            base = NS * ROWS_PER_SUB
            pltpu.sync_copy(acc.at[pl.ds(base, N - base)],
                            out_acc.at[c, pl.ds(base, N - base)])
            pltpu.sync_copy(cnt, out_cnt.at[c])

    return pl.kernel(
        body,
        mesh=_mesh,
        out_type=[
            jax.ShapeDtypeStruct((NC, N, D), jnp.float32),
            jax.ShapeDtypeStruct((NC, N_ACC), jnp.float32),
        ],
        scratch_types=[
            pltpu.VMEM((CPP + 1, CHUNK), jnp.int32),
            pltpu.VMEM((CPP, CHUNK), jnp.int32),
            pltpu.VMEM((2, CHUNK, D), jnp.float32),
            pltpu.VMEM((CHUNK,), jnp.float32),
            pltpu.VMEM_SHARED((N_ACC, D), jnp.float32),
            pltpu.VMEM_SHARED((N_ACC,), jnp.float32),
            pltpu.SemaphoreType.DMA,
            pltpu.SemaphoreType.DMA,
        ],
    )


_agg_cnt = _make_agg(True)

CHUNKS_PS = E_PAD // (NS * CHUNK)
PHASES2 = CHUNKS_PS // CPP


@functools.partial(
    pl.kernel,
    mesh=_mesh,
    out_type=[jax.ShapeDtypeStruct((NC, N, D), jnp.float32)],
    scratch_types=[
        pltpu.VMEM((CPP + 1, CHUNK), jnp.int32),
        pltpu.VMEM((CPP, CHUNK), jnp.int32),
        pltpu.VMEM((2, CHUNK, D), jnp.float32),
        pltpu.VMEM_SHARED((N_ACC, D), jnp.float32),
        pltpu.SemaphoreType.DMA,
        pltpu.SemaphoreType.DMA,
    ],
)
def _agg2(ta, tb, srcr, dstr, z2, out_acc, idx_s, idx_d, rows, acc,
          gsem0, gsem1):
    c = lax.axis_index("c")
    s = lax.axis_index("s")
    gsems = (gsem0, gsem1)

    rps = N_ACC // NS
    pltpu.sync_copy(z2.at[pl.ds(s * rps, rps)], acc.at[pl.ds(s * rps, rps)])
    plsc.subcore_barrier()

    def run(table):
        def _gather_start(cj, b):
            pltpu.async_copy(table.at[idx_s.at[cj]], rows.at[b], gsems[b])

        def _gather_wait(b):
            pltpu.make_async_copy(table.at[idx_s.at[0]], rows.at[b],
                                  gsems[b]).wait()

        for p in range(PHASES2):
            pltpu.sync_copy(srcr.at[s, pl.ds(p * CPP, CPP)],
                            idx_s.at[pl.ds(0, CPP)])
            nxt = (p + 1) * CPP if p + 1 < PHASES2 else 0
            pltpu.sync_copy(srcr.at[s, nxt], idx_s.at[CPP])
            pltpu.sync_copy(dstr.at[s, pl.ds(p * CPP, CPP)], idx_d)

            _gather_start(0, 0)

            def loop(j, carry):
                for b in range(4):
                    cj = 4 * j + b
                    _gather_start(cj + 1, 1 - b % 2)
                    _gather_wait(b % 2)
                    pltpu.sync_copy(rows.at[b % 2], acc.at[idx_d.at[cj]],
                                    add=True)
                return carry

            lax.fori_loop(0, CPP // 4, loop, 0)
            _gather_wait(0)

    @pl.when(c == 0)
    def _():
        run(ta)

    @pl.when(c == 1)
    def _():
        run(tb)

    plsc.subcore_barrier()

    pltpu.sync_copy(acc.at[pl.ds(s * ROWS_PER_SUB, ROWS_PER_SUB)],
                    out_acc.at[c, pl.ds(s * ROWS_PER_SUB, ROWS_PER_SUB)])

    @pl.when(s == 0)
    def _():
        base = NS * ROWS_PER_SUB
        pltpu.sync_copy(acc.at[pl.ds(base, N - base)],
                        out_acc.at[c, pl.ds(base, N - base)])


BM = 5000


def _xw_body(x, w, b, out):
    out[...] = (jnp.dot(x[...], w[...], preferred_element_type=jnp.float32)
                + b[...])


def _xw(x, w, b, k):
    return pl.pallas_call(
        _xw_body,
        grid=(N // BM,),
        in_specs=[
            pl.BlockSpec((BM, k), lambda i: (i, 0)),
            pl.BlockSpec((k, H1), lambda i: (0, 0)),
            pl.BlockSpec((1, H1), lambda i: (0, 0)),
        ],
        out_specs=[pl.BlockSpec((BM, H1), lambda i: (i, 0))],
        out_shape=[jax.ShapeDtypeStruct((N, H1), jnp.float32)],
    )(x, w, b)[0]


def _l1_body(aggp, cntp, xr, w1l, h1a, h1b):
    cnt = cntp[0, :, 0] + cntp[1, :, 0]
    inv = 1.0 / jnp.maximum(cnt, 1.0)
    mean = (aggp[0] + aggp[1]) * inv[:, None]
    h = jnp.dot(mean, w1l[...], preferred_element_type=jnp.float32)
    h = jnp.maximum(h + xr[...], 0.0)
    h1a[...] = h[:, :D]
    h1b[...] = h[:, D:]


def _l1(aggp, cntp, xr, w1l):
    return pl.pallas_call(
        _l1_body,
        grid=(N // BM,),
        in_specs=[
            pl.BlockSpec((NC, BM, D), lambda i: (0, i, 0)),
            pl.BlockSpec((NC, BM, 1), lambda i: (0, i, 0)),
            pl.BlockSpec((BM, H1), lambda i: (i, 0)),
            pl.BlockSpec((D, H1), lambda i: (0, 0)),
        ],
        out_specs=[pl.BlockSpec((BM, D), lambda i: (i, 0))] * 2,
        out_shape=[jax.ShapeDtypeStruct((N, D), jnp.float32)] * 2,
    )(aggp, cntp, xr, w1l)


def _tpre_body(h1a, h1b, w2ra, w2rb, b2, t):
    t[...] = (jnp.dot(h1a[...], w2ra[...], preferred_element_type=jnp.float32)
              + jnp.dot(h1b[...], w2rb[...],
                        preferred_element_type=jnp.float32)
              + b2[...])


def _tpre(h1a, h1b, w2ra, w2rb, b2):
    return pl.pallas_call(
        _tpre_body,
        grid=(N // BM,),
        in_specs=[
            pl.BlockSpec((BM, D), lambda i: (i, 0)),
            pl.BlockSpec((BM, D), lambda i: (i, 0)),
            pl.BlockSpec((D, H2), lambda i: (0, 0)),
            pl.BlockSpec((D, H2), lambda i: (0, 0)),
            pl.BlockSpec((1, H2), lambda i: (0, 0)),
        ],
        out_specs=[pl.BlockSpec((BM, H2), lambda i: (i, 0))],
        out_shape=[jax.ShapeDtypeStruct((N, H2), jnp.float32)],
    )(h1a, h1b, w2ra, w2rb, b2)[0]


def _tail_body(a2, cntp, t, w2la, w2lb, wl1, bl1, wl2, bl2, out):
    cnt = cntp[0, :, 0] + cntp[1, :, 0]
    inv = 1.0 / jnp.maximum(cnt, 1.0)
    m2a = a2[0] * inv[:, None]
    m2b = a2[1] * inv[:, None]
    h2 = (jnp.dot(m2a, w2la[...], preferred_element_type=jnp.float32)
          + jnp.dot(m2b, w2lb[...], preferred_element_type=jnp.float32)
          + t[...])
    u = jnp.maximum(
        jnp.dot(h2, wl1[...], preferred_element_type=jnp.float32) + bl1[...],
        0.0)
    o = jnp.maximum(
        jnp.dot(u, wl2[...], preferred_element_type=jnp.float32) + bl2[...],
        0.0)
    out[...] = o


def _tail(a2, cntp, t, w2la, w2lb, wl1, bl1, wl2, bl2):
    return pl.pallas_call(
        _tail_body,
        grid=(N // BM,),
        in_specs=[
            pl.BlockSpec((NC, BM, D), lambda i: (0, i, 0)),
            pl.BlockSpec((NC, BM, 1), lambda i: (0, i, 0)),
            pl.BlockSpec((BM, H2), lambda i: (i, 0)),
            pl.BlockSpec((D, H2), lambda i: (0, 0)),
            pl.BlockSpec((D, H2), lambda i: (0, 0)),
            pl.BlockSpec((H2, H3), lambda i: (0, 0)),
            pl.BlockSpec((1, H3), lambda i: (0, 0)),
            pl.BlockSpec((H3, 1), lambda i: (0, 0)),
            pl.BlockSpec((1, 1), lambda i: (0, 0)),
        ],
        out_specs=[pl.BlockSpec((BM, 1), lambda i: (i, 0))],
        out_shape=[jax.ShapeDtypeStruct((N, 1), jnp.float32)],
    )(a2, cntp, t, w2la, w2lb, wl1, bl1, wl2, bl2)[0]


def kernel(x, edge_index, W1l, b1, W1r, W2l, b2, W2r, Wlin1, blin1, Wlin2,
           blin2):
    src = edge_index[0].astype(jnp.int32)
    dst = edge_index[1].astype(jnp.int32)
    pad = E_PAD - E
    psrc = jnp.arange(pad, dtype=jnp.int32) % N
    pdst = N + (jnp.arange(pad, dtype=jnp.int32) % PAD_ROWS)
    src_p = jnp.concatenate([src, psrc])
    dst_p = jnp.concatenate([dst, pdst])
    srcr = src_p.reshape(NW, CHUNKS_PW, CHUNK)
    dstr = dst_p.reshape(NW, CHUNKS_PW, CHUNK)
    srcr2 = src_p.reshape(NS, CHUNKS_PS, CHUNK)
    dstr2 = dst_p.reshape(NS, CHUNKS_PS, CHUNK)
    z2 = jnp.zeros((N_ACC, D), jnp.float32)
    z1 = jnp.zeros((N_ACC,), jnp.float32)

    agg1, cnt_full = _agg_cnt(x, srcr, dstr, z2, z1)
    xr = _xw(x, W1r, b1.reshape(1, H1), D)
    cnt = cnt_full[:, :N]
    h1a, h1b = _l1(agg1, cnt[..., None], xr, W1l)
    agg2 = _agg2(h1a, h1b, srcr2, dstr2, z2)[0]
    t = _tpre(h1a, h1b, W2r[:D], W2r[D:],
              b2.reshape(1, H2))
    return _tail(agg2, cnt[..., None], t, W2l[:D], W2l[D:],
                 Wlin1, blin1.reshape(1, H3), Wlin2, blin2.reshape(1, 1))


# speedup vs baseline: 1.0020x; 1.0020x over previous
import functools

import jax
import jax.numpy as jnp
from jax import lax
from jax.experimental import pallas as pl
from jax.experimental.pallas import tpu as pltpu
from jax.experimental.pallas import tpu_sc as plsc

N = 10000
E = 320000
D = 128
H1 = 256
H2 = 256
H3 = 128

NC, NS, LANES = 2, 16, 16
NW = NC * NS
CHUNK = 128
PHASES = 2
CPP = 40
CHUNKS_PW = PHASES * CPP
E_PAD = NW * CHUNKS_PW * CHUNK
PAD_ROWS = 112
N_ACC = N + PAD_ROWS
ROWS_PER_SUB = 624

_mesh = plsc.VectorSubcoreMesh(core_axis_name="c", subcore_axis_name="s")


def _make_agg(with_count):
    def body(table, srcr, dstr, z2, z1, out_acc, out_cnt,
             idx_s, idx_d, rows, ones_v, acc, cnt, gsem0, gsem1):
        c = lax.axis_index("c")
        s = lax.axis_index("s")
        w = s * NC + c
        gsems = (gsem0, gsem1)

        if with_count:
            for k in range(CHUNK // LANES):
                ones_v[pl.ds(k * LANES, LANES)] = jnp.ones((LANES,),
                                                           jnp.float32)

        rps = N_ACC // NS
        pltpu.sync_copy(z2.at[pl.ds(s * rps, rps)],
                        acc.at[pl.ds(s * rps, rps)])

        @pl.when(s == 0)
        def _():
            pltpu.sync_copy(z1, cnt)

        plsc.subcore_barrier()

        def _gather_start(cj, b):
            pltpu.async_copy(table.at[idx_s.at[cj]], rows.at[b], gsems[b])

        def _gather_wait(b):
            pltpu.make_async_copy(table.at[idx_s.at[0]], rows.at[b],
                                  gsems[b]).wait()

        for p in range(PHASES):
            pltpu.sync_copy(srcr.at[w, pl.ds(p * CPP, CPP)],
                            idx_s.at[pl.ds(0, CPP)])
            nxt = (p + 1) * CPP if p + 1 < PHASES else 0
            pltpu.sync_copy(srcr.at[w, nxt], idx_s.at[CPP])
            pltpu.sync_copy(dstr.at[w, pl.ds(p * CPP, CPP)], idx_d)

            _gather_start(0, 0)

            def loop(j, carry):
                for b in range(4):
                    cj = 4 * j + b
                    _gather_start(cj + 1, 1 - b % 2)
                    if with_count:
                        pltpu.sync_copy(ones_v, cnt.at[idx_d.at[cj]],
                                        add=True)
                    _gather_wait(b % 2)
                    pltpu.sync_copy(rows.at[b % 2], acc.at[idx_d.at[cj]],
                                    add=True)
                return carry

            lax.fori_loop(0, CPP // 4, loop, 0)
            _gather_wait(0)

        plsc.subcore_barrier()

        pltpu.sync_copy(acc.at[pl.ds(s * ROWS_PER_SUB, ROWS_PER_SUB)],
                        out_acc.at[c, pl.ds(s * ROWS_PER_SUB, ROWS_PER_SUB)])

        @pl.when(s == 0)
        def _():
            base = NS * ROWS_PER_SUB
            pltpu.sync_copy(acc.at[pl.ds(base, N - base)],
                            out_acc.at[c, pl.ds(base, N - base)])
            pltpu.sync_copy(cnt, out_cnt.at[c])

    return pl.kernel(
        body,
        mesh=_mesh,
        out_type=[
            jax.ShapeDtypeStruct((NC, N, D), jnp.float32),
            jax.ShapeDtypeStruct((NC, N_ACC), jnp.float32),
        ],
        scratch_types=[
            pltpu.VMEM((CPP + 1, CHUNK), jnp.int32),
            pltpu.VMEM((CPP, CHUNK), jnp.int32),
            pltpu.VMEM((2, CHUNK, D), jnp.float32),
            pltpu.VMEM((CHUNK,), jnp.float32),
            pltpu.VMEM_SHARED((N_ACC, D), jnp.float32),
            pltpu.VMEM_SHARED((N_ACC,), jnp.float32),
            pltpu.SemaphoreType.DMA,
            pltpu.SemaphoreType.DMA,
        ],
    )


_agg_cnt = _make_agg(True)

CHUNKS_PS = E_PAD // (NS * CHUNK)
PHASES2 = CHUNKS_PS // CPP


@functools.partial(
    pl.kernel,
    mesh=_mesh,
    out_type=[jax.ShapeDtypeStruct((NC, N, D), jnp.float32)],
    scratch_types=[
        pltpu.VMEM((CPP + 1, CHUNK), jnp.int32),
        pltpu.VMEM((CPP, CHUNK), jnp.int32),
        pltpu.VMEM((2, CHUNK, D), jnp.float32),
        pltpu.VMEM_SHARED((N_ACC, D), jnp.float32),
        pltpu.SemaphoreType.DMA,
        pltpu.SemaphoreType.DMA,
    ],
)
def _agg2(ta, tb, srcr, dstr, z2, out_acc, idx_s, idx_d, rows, acc,
          gsem0, gsem1):
    c = lax.axis_index("c")
    s = lax.axis_index("s")
    gsems = (gsem0, gsem1)

    rps = N_ACC // NS
    pltpu.sync_copy(z2.at[pl.ds(s * rps, rps)], acc.at[pl.ds(s * rps, rps)])
    plsc.subcore_barrier()

    def run(table):
        def _gather_start(cj, b):
            pltpu.async_copy(table.at[idx_s.at[cj]], rows.at[b], gsems[b])

        def _gather_wait(b):
            pltpu.make_async_copy(table.at[idx_s.at[0]], rows.at[b],
                                  gsems[b]).wait()

        for p in range(PHASES2):
            pltpu.sync_copy(srcr.at[s, pl.ds(p * CPP, CPP)],
                            idx_s.at[pl.ds(0, CPP)])
            nxt = (p + 1) * CPP if p + 1 < PHASES2 else 0
            pltpu.sync_copy(srcr.at[s, nxt], idx_s.at[CPP])
            pltpu.sync_copy(dstr.at[s, pl.ds(p * CPP, CPP)], idx_d)

            _gather_start(0, 0)

            def loop(j, carry):
                for b in range(4):
                    cj = 4 * j + b
                    _gather_start(cj + 1, 1 - b % 2)
                    _gather_wait(b % 2)
                    pltpu.sync_copy(rows.at[b % 2], acc.at[idx_d.at[cj]],
                                    add=True)
                return carry

            lax.fori_loop(0, CPP // 4, loop, 0)
            _gather_wait(0)

    @pl.when(c == 0)
    def _():
        run(ta)

    @pl.when(c == 1)
    def _():
        run(tb)

    plsc.subcore_barrier()

    pltpu.sync_copy(acc.at[pl.ds(s * ROWS_PER_SUB, ROWS_PER_SUB)],
                    out_acc.at[c, pl.ds(s * ROWS_PER_SUB, ROWS_PER_SUB)])

    @pl.when(s == 0)
    def _():
        base = NS * ROWS_PER_SUB
        pltpu.sync_copy(acc.at[pl.ds(base, N - base)],
                        out_acc.at[c, pl.ds(base, N - base)])


BM = 2000


def _xw_body(x, w, b, out):
    out[...] = (jnp.dot(x[...], w[...], preferred_element_type=jnp.float32)
                + b[...])


def _xw(x, w, b, k):
    return pl.pallas_call(
        _xw_body,
        grid=(N // BM,),
        in_specs=[
            pl.BlockSpec((BM, k), lambda i: (i, 0)),
            pl.BlockSpec((k, H1), lambda i: (0, 0)),
            pl.BlockSpec((1, H1), lambda i: (0, 0)),
        ],
        out_specs=[pl.BlockSpec((BM, H1), lambda i: (i, 0))],
        out_shape=[jax.ShapeDtypeStruct((N, H1), jnp.float32)],
    )(x, w, b)[0]


def _l1_body(aggp, cntp, xr, w1l, h1a, h1b):
    cnt = cntp[0, :, 0] + cntp[1, :, 0]
    inv = 1.0 / jnp.maximum(cnt, 1.0)
    mean = (aggp[0] + aggp[1]) * inv[:, None]
    h = jnp.dot(mean, w1l[...], preferred_element_type=jnp.float32)
    h = jnp.maximum(h + xr[...], 0.0)
    h1a[...] = h[:, :D]
    h1b[...] = h[:, D:]


def _l1(aggp, cntp, xr, w1l):
    return pl.pallas_call(
        _l1_body,
        grid=(N // BM,),
        in_specs=[
            pl.BlockSpec((NC, BM, D), lambda i: (0, i, 0)),
            pl.BlockSpec((NC, BM, 1), lambda i: (0, i, 0)),
            pl.BlockSpec((BM, H1), lambda i: (i, 0)),
            pl.BlockSpec((D, H1), lambda i: (0, 0)),
        ],
        out_specs=[pl.BlockSpec((BM, D), lambda i: (i, 0))] * 2,
        out_shape=[jax.ShapeDtypeStruct((N, D), jnp.float32)] * 2,
    )(aggp, cntp, xr, w1l)


def _tpre_body(h1a, h1b, w2ra, w2rb, b2, t):
    t[...] = (jnp.dot(h1a[...], w2ra[...], preferred_element_type=jnp.float32)
              + jnp.dot(h1b[...], w2rb[...],
                        preferred_element_type=jnp.float32)
              + b2[...])


def _tpre(h1a, h1b, w2ra, w2rb, b2):
    return pl.pallas_call(
        _tpre_body,
        grid=(N // BM,),
        in_specs=[
            pl.BlockSpec((BM, D), lambda i: (i, 0)),
            pl.BlockSpec((BM, D), lambda i: (i, 0)),
            pl.BlockSpec((D, H2), lambda i: (0, 0)),
            pl.BlockSpec((D, H2), lambda i: (0, 0)),
            pl.BlockSpec((1, H2), lambda i: (0, 0)),
        ],
        out_specs=[pl.BlockSpec((BM, H2), lambda i: (i, 0))],
        out_shape=[jax.ShapeDtypeStruct((N, H2), jnp.float32)],
    )(h1a, h1b, w2ra, w2rb, b2)[0]


def _tail_body(a2, cntp, t, w2la, w2lb, wl1, bl1, wl2, bl2, out):
    cnt = cntp[0, :, 0] + cntp[1, :, 0]
    inv = 1.0 / jnp.maximum(cnt, 1.0)
    m2a = a2[0] * inv[:, None]
    m2b = a2[1] * inv[:, None]
    h2 = (jnp.dot(m2a, w2la[...], preferred_element_type=jnp.float32)
          + jnp.dot(m2b, w2lb[...], preferred_element_type=jnp.float32)
          + t[...])
    u = jnp.maximum(
        jnp.dot(h2, wl1[...], preferred_element_type=jnp.float32) + bl1[...],
        0.0)
    o = jnp.maximum(
        jnp.dot(u, wl2[...], preferred_element_type=jnp.float32) + bl2[...],
        0.0)
    out[...] = o


def _tail(a2, cntp, t, w2la, w2lb, wl1, bl1, wl2, bl2):
    return pl.pallas_call(
        _tail_body,
        grid=(N // BM,),
        in_specs=[
            pl.BlockSpec((NC, BM, D), lambda i: (0, i, 0)),
            pl.BlockSpec((NC, BM, 1), lambda i: (0, i, 0)),
            pl.BlockSpec((BM, H2), lambda i: (i, 0)),
            pl.BlockSpec((D, H2), lambda i: (0, 0)),
            pl.BlockSpec((D, H2), lambda i: (0, 0)),
            pl.BlockSpec((H2, H3), lambda i: (0, 0)),
            pl.BlockSpec((1, H3), lambda i: (0, 0)),
            pl.BlockSpec((H3, 1), lambda i: (0, 0)),
            pl.BlockSpec((1, 1), lambda i: (0, 0)),
        ],
        out_specs=[pl.BlockSpec((BM, 1), lambda i: (i, 0))],
        out_shape=[jax.ShapeDtypeStruct((N, 1), jnp.float32)],
    )(a2, cntp, t, w2la, w2lb, wl1, bl1, wl2, bl2)[0]


def kernel(x, edge_index, W1l, b1, W1r, W2l, b2, W2r, Wlin1, blin1, Wlin2,
           blin2):
    src = edge_index[0].astype(jnp.int32)
    dst = edge_index[1].astype(jnp.int32)
    pad = E_PAD - E
    psrc = jnp.arange(pad, dtype=jnp.int32) % N
    pdst = N + (jnp.arange(pad, dtype=jnp.int32) % PAD_ROWS)
    src_p = jnp.concatenate([src, psrc])
    dst_p = jnp.concatenate([dst, pdst])
    srcr = src_p.reshape(NW, CHUNKS_PW, CHUNK)
    dstr = dst_p.reshape(NW, CHUNKS_PW, CHUNK)
    srcr2 = src_p.reshape(NS, CHUNKS_PS, CHUNK)
    dstr2 = dst_p.reshape(NS, CHUNKS_PS, CHUNK)
    z2 = jnp.zeros((N_ACC, D), jnp.float32)
    z1 = jnp.zeros((N_ACC,), jnp.float32)

    agg1, cnt_full = _agg_cnt(x, srcr, dstr, z2, z1)
    xr = _xw(x, W1r, b1.reshape(1, H1), D)
    cnt = cnt_full[:, :N]
    h1a, h1b = _l1(agg1, cnt[..., None], xr, W1l)
    agg2 = _agg2(h1a, h1b, srcr2, dstr2, z2)[0]
    t = _tpre(h1a, h1b, W2r[:D], W2r[D:],
              b2.reshape(1, H2))
    return _tail(agg2, cnt[..., None], t, W2l[:D], W2l[D:],
                 Wlin1, blin1.reshape(1, H3), Wlin2, blin2.reshape(1, 1))
